# baseline (speedup 1.0000x reference)
import jax
import jax.numpy as jnp
import numpy as np
from jax.experimental import pallas as pl
from jax.experimental.pallas import tpu as pltpu

_PI = float(np.pi)


def _fps_body(px_ref, py_ref, pz_ref, ox_ref, oy_ref, oz_ref, *, m, s):
    X = px_ref[...]
    Y = py_ref[...]
    Z = pz_ref[...]
    sm = max(m // 128, 1)
    idxg = (jax.lax.broadcasted_iota(jnp.int32, (s, 128), 0) * 128
            + jax.lax.broadcasted_iota(jnp.int32, (s, 128), 1))
    idxm = (jax.lax.broadcasted_iota(jnp.int32, (sm, 128), 0) * 128
            + jax.lax.broadcasted_iota(jnp.int32, (sm, 128), 1))
    qx0 = px_ref[0, 0]
    qy0 = py_ref[0, 0]
    qz0 = pz_ref[0, 0]
    dx = X - qx0
    dy = Y - qy0
    dz = Z - qz0
    dists0 = (dx * dx + dy * dy) + dz * dz
    zf = jnp.zeros((sm, 128), jnp.float32)
    selx0 = jnp.where(idxm == 0, qx0, zf)
    sely0 = jnp.where(idxm == 0, qy0, zf)
    selz0 = jnp.where(idxm == 0, qz0, zf)

    def body(i, c):
        dists, qx, qy, qz, selx, sely, selz = c
        dx = X - qx
        dy = Y - qy
        dz = Z - qz
        d = (dx * dx + dy * dy) + dz * dz
        dists = jnp.minimum(dists, d)
        mx = jnp.max(dists)
        nxt = jnp.min(jnp.where(dists == mx, idxg, jnp.int32(2 ** 30)))
        mask = idxg == nxt
        nqx = jnp.sum(jnp.where(mask, X, 0.0))
        nqy = jnp.sum(jnp.where(mask, Y, 0.0))
        nqz = jnp.sum(jnp.where(mask, Z, 0.0))
        mi = idxm == i
        selx = jnp.where(mi, nqx, selx)
        sely = jnp.where(mi, nqy, sely)
        selz = jnp.where(mi, nqz, selz)
        return (dists, nqx, nqy, nqz, selx, sely, selz)

    c = jax.lax.fori_loop(1, m, body,
                          (dists0, qx0, qy0, qz0, selx0, sely0, selz0))
    ox_ref[...] = c[4]
    oy_ref[...] = c[5]
    oz_ref[...] = c[6]


def _fps(px, py, pz, m):
    s = px.shape[0]
    sm = max(m // 128, 1)
    import functools
    body = functools.partial(_fps_body, m=m, s=s)
    shp = jax.ShapeDtypeStruct((sm, 128), jnp.float32)
    return pl.pallas_call(body, out_shape=(shp, shp, shp))(px, py, pz)


def _knn_body(qx_ref, qy_ref, qz_ref, px_ref, py_ref, pz_ref,
              pdx_ref, pdy_ref, pdz_ref, col_ref, d_ref, n2_ref,
              *, s, qper, smq):
    step = pl.program_id(0)

    @pl.when(step == 0)
    def _():
        X = px_ref[...]
        Y = py_ref[...]
        Z = pz_ref[...]
        n2_ref[...] = (X * X + Y * Y) + Z * Z

    X = px_ref[...]
    Y = py_ref[...]
    Z = pz_ref[...]
    Xb = X.astype(jnp.bfloat16).astype(jnp.float32)
    Yb = Y.astype(jnp.bfloat16).astype(jnp.float32)
    Zb = Z.astype(jnp.bfloat16).astype(jnp.float32)
    n2 = n2_ref[...]
    idxq = (jax.lax.broadcasted_iota(jnp.int32, (smq, 128), 0) * 128
            + jax.lax.broadcasted_iota(jnp.int32, (smq, 128), 1))
    iota_r = jax.lax.broadcasted_iota(jnp.int32, (s, 1), 0)
    iota_l = jax.lax.broadcasted_iota(jnp.int32, (1, 128), 1)
    iota32 = jax.lax.broadcasted_iota(jnp.int32, (1, 32), 1)
    QX = qx_ref[...]
    QY = qy_ref[...]
    QZ = qz_ref[...]

    rm = [None] * qper
    qxs = [None] * qper
    qys = [None] * qper
    qzs = [None] * qper
    for q in range(qper):
        g = step * qper + q
        qmask = idxq == g
        qx = jnp.sum(jnp.where(qmask, QX, 0.0))
        qy = jnp.sum(jnp.where(qmask, QY, 0.0))
        qz = jnp.sum(jnp.where(qmask, QZ, 0.0))
        qxs[q], qys[q], qzs[q] = qx, qy, qz
        ny = (qx * qx + qy * qy) + qz * qz
        qxb = qx.astype(jnp.bfloat16).astype(jnp.float32)
        qyb = qy.astype(jnp.bfloat16).astype(jnp.float32)
        qzb = qz.astype(jnp.bfloat16).astype(jnp.float32)
        d = (ny + n2) - 2.0 * ((qxb * Xb + qyb * Yb) + qzb * Zb)
        d_ref[q] = d
        rm[q] = jnp.min(d, axis=1, keepdims=True)

    pdxa = [jnp.zeros((1, 32), jnp.float32) for _ in range(qper)]
    pdya = [jnp.zeros((1, 32), jnp.float32) for _ in range(qper)]
    pdza = [jnp.zeros((1, 32), jnp.float32) for _ in range(qper)]
    cola = [jnp.zeros((1, 32), jnp.int32) for _ in range(qper)]
    inf = jnp.float32(np.inf)
    for j in range(32):
        for q in range(qper):
            mval = jnp.min(rm[q])
            r = jnp.min(jnp.where(rm[q] == mval, iota_r, jnp.int32(2 ** 30)))
            row = d_ref[q, pl.ds(r, 1), :]
            l = jnp.min(jnp.where(row == mval, iota_l, jnp.int32(2 ** 30)))
            lmask = iota_l == l
            xrow = px_ref[pl.ds(r, 1), :]
            yrow = py_ref[pl.ds(r, 1), :]
            zrow = pz_ref[pl.ds(r, 1), :]
            nx = jnp.sum(jnp.where(lmask, xrow, 0.0))
            nyv = jnp.sum(jnp.where(lmask, yrow, 0.0))
            nz = jnp.sum(jnp.where(lmask, zrow, 0.0))
            jm = iota32 == j
            pdxa[q] = jnp.where(jm, nx - qxs[q], pdxa[q])
            pdya[q] = jnp.where(jm, nyv - qys[q], pdya[q])
            pdza[q] = jnp.where(jm, nz - qzs[q], pdza[q])
            cola[q] = jnp.where(jm, r * 128 + l, cola[q])
            newrow = jnp.where(lmask, inf, row)
            d_ref[q, pl.ds(r, 1), :] = newrow
            rm[q] = jnp.where(iota_r == r, jnp.min(newrow), rm[q])

    for q in range(qper):
        pdx_ref[0, q, :] = pdxa[q][0, :]
        pdy_ref[0, q, :] = pdya[q][0, :]
        pdz_ref[0, q, :] = pdza[q][0, :]
        col_ref[0, q, :] = cola[q][0, :]


def _knn(qx, qy, qz, px, py, pz, m, qper=4):
    s = px.shape[0]
    smq = qx.shape[0]
    import functools
    body = functools.partial(_knn_body, s=s, qper=qper, smq=smq)
    shp = jax.ShapeDtypeStruct((m // qper, qper, 32), jnp.float32)
    shpi = jax.ShapeDtypeStruct((m // qper, qper, 32), jnp.int32)
    full_q = pl.BlockSpec((smq, 128), lambda i: (0, 0))
    full_p = pl.BlockSpec((s, 128), lambda i: (0, 0))
    outb = pl.BlockSpec((1, qper, 32), lambda i: (i, 0, 0))
    return pl.pallas_call(
        body,
        grid=(m // qper,),
        in_specs=[full_q, full_q, full_q, full_p, full_p, full_p],
        out_specs=[outb, outb, outb, outb],
        out_shape=(shp, shp, shp, shpi),
        scratch_shapes=[pltpu.VMEM((qper, s, 128), jnp.float32),
                        pltpu.VMEM((s, 128), jnp.float32)],
    )(qx, qy, qz, px, py, pz)


def _posenc_feat(pdx, pdy, pdz, n):
    e = jax.lax.broadcasted_iota(jnp.int32, (1, 64), 1)
    k = jnp.maximum(e - 3, 0)
    c = k // 20
    lf = (k % 20) // 2
    is_sin = (k % 2) == 0
    is_coord = e < 3
    coordid = jnp.where(is_coord, e, jnp.minimum(c, 2))
    freq = (jnp.int32(1) << lf).astype(jnp.float32) * _PI
    raw = jnp.where(coordid == 0, pdx,
                    jnp.where(coordid == 1, pdy, pdz))
    scaled = raw * jnp.where(is_coord, jnp.float32(1.0), freq)
    sv = jnp.sin(scaled)
    cv = jnp.cos(scaled)
    return jnp.where(is_coord, raw, jnp.where(is_sin, sv, cv))


def _sa1_body(pdx_ref, pdy_ref, pdz_ref, w0_ref, b0_ref, w1_ref, b1_ref,
              w2_ref, b2_ref, o1_ref):
    n = pdx_ref.shape[0]
    feat = _posenc_feat(pdx_ref[...], pdy_ref[...], pdz_ref[...], n)
    h = jax.lax.dot_general(feat, w0_ref[...], (((1,), (0,)), ((), ())),
                            preferred_element_type=jnp.float32) + b0_ref[...]
    h = jnp.maximum(h, 0.0)
    h = jax.lax.dot_general(h, w1_ref[...], (((1,), (0,)), ((), ())),
                            preferred_element_type=jnp.float32) + b1_ref[...]
    h = jnp.maximum(h, 0.0)
    h = jax.lax.dot_general(h, w2_ref[...], (((1,), (0,)), ((), ())),
                            preferred_element_type=jnp.float32) + b2_ref[...]
    hm = jnp.max(h.reshape(n // 32, 32, 128), axis=1)
    o1_ref[...] = hm


def _sa1_edge(pdxf, pdyf, pdzf, w0p, b0, w1, b1, w2, b2):
    ne = pdxf.shape[0]
    tile = 4096
    grid = ne // tile
    pdb = pl.BlockSpec((tile, 1), lambda i: (i, 0))
    wfull = lambda a: pl.BlockSpec(a.shape, lambda i: (0,) * a.ndim)
    return pl.pallas_call(
        _sa1_body,
        grid=(grid,),
        in_specs=[pdb, pdb, pdb, wfull(w0p), wfull(b0), wfull(w1),
                  wfull(b1), wfull(w2), wfull(b2)],
        out_specs=pl.BlockSpec((tile // 32, 128), lambda i: (i, 0)),
        out_shape=jax.ShapeDtypeStruct((ne // 32, 128), jnp.float32),
    )(pdxf, pdyf, pdzf, w0p, b0, w1, b1, w2, b2)


def _glob_body(x_ref, w0_ref, b0_ref, w1_ref, b1_ref, o_ref):
    h = jax.lax.dot_general(x_ref[...], w0_ref[...], (((1,), (0,)), ((), ())),
                            preferred_element_type=jnp.float32) + b0_ref[...]
    h = jnp.maximum(h, 0.0)
    h = jax.lax.dot_general(h, w1_ref[...], (((1,), (0,)), ((), ())),
                            preferred_element_type=jnp.float32) + b1_ref[...]
    o_ref[...] = h


def _glob(x, w0, b0, w1, b1):
    m = x.shape[0]
    return pl.pallas_call(
        _glob_body,
        out_shape=jax.ShapeDtypeStruct((m, w1.shape[1]), jnp.float32),
    )(x, w0, b0, w1, b1)


def _sa2_body(col_ref, pdx_ref, pdy_ref, pdz_ref, x1_ref,
              w0a_ref, w0b_ref, b0_ref, w1_ref, b1_ref, w2_ref, b2_ref,
              o2_ref):
    n = col_ref.shape[0]
    col = col_ref[...]
    og = jnp.zeros((n, 256), jnp.float32)
    for cb in range(8):
        iota_c = (jax.lax.broadcasted_iota(jnp.int32, (1, 128), 1)
                  + cb * 128)
        ohc = jnp.where(col == iota_c, 1.0, 0.0)
        og = og + jax.lax.dot_general(
            ohc, x1_ref[pl.ds(cb * 128, 128), :], (((1,), (0,)), ((), ())),
            preferred_element_type=jnp.float32)
    feat = _posenc_feat(pdx_ref[...], pdy_ref[...], pdz_ref[...], n)
    h = (jax.lax.dot_general(og, w0a_ref[...], (((1,), (0,)), ((), ())),
                             preferred_element_type=jnp.float32)
         + jax.lax.dot_general(feat, w0b_ref[...], (((1,), (0,)), ((), ())),
                               preferred_element_type=jnp.float32)
         + b0_ref[...])
    h = jnp.maximum(h, 0.0)
    h = jax.lax.dot_general(h, w1_ref[...], (((1,), (0,)), ((), ())),
                            preferred_element_type=jnp.float32) + b1_ref[...]
    h = jnp.maximum(h, 0.0)
    h = jax.lax.dot_general(h, w2_ref[...], (((1,), (0,)), ((), ())),
                            preferred_element_type=jnp.float32) + b2_ref[...]
    hm = jnp.max(h.reshape(n // 32, 32, 512), axis=1)
    o2_ref[...] = hm


def _sa2_edge(colf, pd2xf, pd2yf, pd2zf, x1, w0a, w0bp, b0, w1, b1, w2, b2):
    ne = colf.shape[0]
    tile = 1024
    grid = ne // tile
    cb = pl.BlockSpec((tile, 1), lambda i: (i, 0))
    wfull = lambda a: pl.BlockSpec(a.shape, lambda i: (0,) * a.ndim)
    return pl.pallas_call(
        _sa2_body,
        grid=(grid,),
        in_specs=[cb, cb, cb, cb, wfull(x1), wfull(w0a), wfull(w0bp),
                  wfull(b0), wfull(w1), wfull(b1), wfull(w2), wfull(b2)],
        out_specs=pl.BlockSpec((tile // 32, 512), lambda i: (i, 0)),
        out_shape=jax.ShapeDtypeStruct((ne // 32, 512), jnp.float32),
    )(colf, pd2xf, pd2yf, pd2zf, x1, w0a, w0bp, b0, w1, b1, w2, b2)


def kernel(pos, s1l0w, s1l0b, s1l1w, s1l1b, s1l2w, s1l2b,
           s1g0w, s1g0b, s1g1w, s1g1b,
           s2l0w, s2l0b, s2l1w, s2l1b, s2l2w, s2l2b,
           s2g0w, s2g0b, s2g1w, s2g1b):
    f = jnp.float32
    px = pos[:, 0].reshape(128, 128)
    py = pos[:, 1].reshape(128, 128)
    pz = pos[:, 2].reshape(128, 128)

    p1x, p1y, p1z = _fps(px, py, pz, 1024)
    pdx, pdy, pdz, _ = _knn(p1x, p1y, p1z, px, py, pz, 1024, qper=4)
    w0p = jnp.pad(s1l0w, ((0, 1), (0, 0)))
    o1 = _sa1_edge(pdx.reshape(32768, 1), pdy.reshape(32768, 1),
                   pdz.reshape(32768, 1), w0p, s1l0b.reshape(1, -1),
                   s1l1w, s1l1b.reshape(1, -1), s1l2w, s1l2b.reshape(1, -1))
    x1 = _glob(o1, s1g0w, s1g0b.reshape(1, -1), s1g1w, s1g1b.reshape(1, -1))

    p2x, p2y, p2z = _fps(p1x, p1y, p1z, 128)
    pd2x, pd2y, pd2z, col2 = _knn(p2x, p2y, p2z, p1x, p1y, p1z, 128, qper=4)
    w0a = s2l0w[:256]
    w0bp = jnp.pad(s2l0w[256:], ((0, 1), (0, 0)))
    o2 = _sa2_edge(col2.reshape(4096, 1), pd2x.reshape(4096, 1),
                   pd2y.reshape(4096, 1), pd2z.reshape(4096, 1), x1,
                   w0a, w0bp, s2l0b.reshape(1, -1), s2l1w,
                   s2l1b.reshape(1, -1), s2l2w, s2l2b.reshape(1, -1))
    x2 = _glob(o2, s2g0w, s2g0b.reshape(1, -1), s2g1w, s2g1b.reshape(1, -1))

    pos2 = jnp.stack([p2x.reshape(128), p2y.reshape(128),
                      p2z.reshape(128)], axis=1)
    return (x2, pos2)
